# baseline (speedup 1.0000x reference)
import functools

import jax
import jax.numpy as jnp
from jax import lax
from jax.experimental import pallas as pl
from jax.experimental.pallas import tpu as pltpu
from jax.experimental.pallas import tpu_sc as plsc

N = 10000
K = 32
C = 128
NW = 32
CPC = 4
CPT = 320
NPAD = NW * CPT
NCHUNK = CPT // CPC
NBUF = 2
IDXC = CPC * K
FCH = C // 16
NPAD16 = NPAD + 16


def _prep_body(feats, ptsB, ptsA, wcomb, wg, bg, q_out, p_out, wcg_out):
    wcg = jnp.dot(wcomb[:], wg[:], preferred_element_type=jnp.float32)
    q_out[:] = (jnp.dot(feats[:], wg[:], preferred_element_type=jnp.float32)
                + jnp.dot(ptsB[:], wcg, preferred_element_type=jnp.float32))
    p_out[:] = jnp.dot(ptsA[:], wcg, preferred_element_type=jnp.float32) + bg[:]
    wcg_out[:] = wcg


_prep = pl.pallas_call(
    _prep_body,
    out_shape=(
        jax.ShapeDtypeStruct((NPAD, C), jnp.float32),
        jax.ShapeDtypeStruct((NPAD, C), jnp.float32),
        jax.ShapeDtypeStruct((8, C), jnp.float32),
    ),
)


def _rsqrt(d2):
    i = plsc.bitcast(d2, jnp.int32)
    i = jnp.int32(0x5F3759DF) - lax.shift_right_logical(i, 1)
    r = plsc.bitcast(i, jnp.float32)
    for _ in range(3):
        r = r * (1.5 - 0.5 * d2 * r * r)
    return r


def _sc_body(q_hbm, p_hbm, x_hbm, y_hbm, z_hbm, idx_hbm, w9_hbm, m_hbm,
             x_v, y_v, z_v, idx_all, qb0, qb1, pb0, pb1, w9v, oall,
             sem0, sem1):
    wid = lax.axis_index("s") * 2 + lax.axis_index("c")
    base_center = wid * CPT

    pltpu.sync_copy(x_hbm, x_v)
    pltpu.sync_copy(y_hbm, y_v)
    pltpu.sync_copy(z_hbm, z_v)
    pltpu.sync_copy(w9_hbm, w9v)
    pltpu.sync_copy(idx_hbm.at[pl.ds(base_center * K, CPT * K)], idx_all)

    bufs = ((qb0, pb0, sem0), (qb1, pb1, sem1))
    wch = [w9v[pl.ds(f * 16, 16)] for f in range(FCH)]

    def fire(ci, qb, pb, sem):
        pltpu.async_copy(q_hbm.at[idx_all.at[pl.ds(ci * IDXC, IDXC)]], qb, sem)
        pltpu.async_copy(p_hbm.at[pl.ds(base_center + ci * CPC, CPC)], pb, sem)

    def drain(ci, qb, pb, sem):
        pltpu.make_async_copy(
            q_hbm.at[idx_all.at[pl.ds(ci * IDXC, IDXC)]], qb, sem).wait()
        pltpu.make_async_copy(
            p_hbm.at[pl.ds(base_center + ci * CPC, CPC)], pb, sem).wait()

    def compute(ci, qb, pb):
        cvec = base_center + ci * CPC
        xc = x_v[pl.ds(cvec, 16)]
        yc = y_v[pl.ds(cvec, 16)]
        zc = z_v[pl.ds(cvec, 16)]
        for cc in range(CPC):
            xi, yi, zi = xc[cc], yc[cc], zc[cc]
            dist = []
            for g in range(2):
                jv = idx_all[pl.ds(ci * IDXC + cc * K + g * 16, 16)]
                dx = xi - plsc.load_gather(x_v, [jv])
                dy = yi - plsc.load_gather(y_v, [jv])
                dz = zi - plsc.load_gather(z_v, [jv])
                d2 = dx * dx + dy * dy + dz * dz
                dist.append(d2 * _rsqrt(jnp.maximum(d2, 1e-24)))
            acc = [None] * FCH
            for k in range(K):
                ds_ = dist[k // 16][k % 16]
                row = cc * K + k
                for f in range(FCH):
                    v = qb[row, pl.ds(f * 16, 16)] + ds_ * wch[f]
                    acc[f] = v if k == 0 else jnp.maximum(acc[f], v)
            lrow = ci * CPC + cc
            for f in range(FCH):
                pv = pb[cc, pl.ds(f * 16, 16)]
                oall[lrow, pl.ds(f * 16, 16)] = jnp.maximum(acc[f] + pv, 0.0)

    for b in range(NBUF):
        fire(b, *bufs[b])

    def outer(cg, carry):
        for b in range(NBUF):
            ci = cg * NBUF + b
            qb, pb, sem = bufs[b]
            drain(ci, qb, pb, sem)
            compute(ci, qb, pb)
            nci = ci + NBUF

            @pl.when(nci < NCHUNK)
            def _():
                fire(nci, qb, pb, sem)
        return carry

    lax.fori_loop(0, NCHUNK // NBUF, outer, 0)
    pltpu.sync_copy(oall, m_hbm.at[pl.ds(base_center, CPT)])


_sc_edge = functools.partial(
    pl.kernel,
    mesh=plsc.VectorSubcoreMesh(core_axis_name="c", subcore_axis_name="s"),
    out_type=jax.ShapeDtypeStruct((NPAD, C), jnp.float32),
    compiler_params=pltpu.CompilerParams(needs_layout_passes=False),
    scratch_types=[
        pltpu.VMEM((NPAD16,), jnp.float32),
        pltpu.VMEM((NPAD16,), jnp.float32),
        pltpu.VMEM((NPAD16,), jnp.float32),
        pltpu.VMEM((CPT * K,), jnp.int32),
        pltpu.VMEM((IDXC, C), jnp.float32),
        pltpu.VMEM((IDXC, C), jnp.float32),
        pltpu.VMEM((CPC, C), jnp.float32),
        pltpu.VMEM((CPC, C), jnp.float32),
        pltpu.VMEM((C,), jnp.float32),
        pltpu.VMEM((CPT, C), jnp.float32),
        pltpu.SemaphoreType.DMA,
        pltpu.SemaphoreType.DMA,
    ],
)(_sc_body)


def _out_body(m, feats, wo, bo, gamma, beta, o_ref):
    z = (jnp.dot(m[:] + feats[:], wo[:], preferred_element_type=jnp.float32)
         + bo[:])
    mu = jnp.mean(z, axis=-1, keepdims=True)
    var = jnp.mean((z - mu) ** 2, axis=-1, keepdims=True)
    o_ref[:] = jnp.maximum(
        (z - mu) / jnp.sqrt(var + 1e-5) * gamma[:] + beta[:], 0.0)


_outk = pl.pallas_call(
    _out_body,
    out_shape=jax.ShapeDtypeStruct((N, C), jnp.float32),
)


def kernel(points, features, grouped_idx, Wpos, bpos, Wg, bg, Wo, bo, gamma, beta):
    pts = points[0]
    feats = features[0]
    idx = grouped_idx[0].astype(jnp.int32).reshape(-1)

    pad = NPAD - N
    feats_p = jnp.pad(feats, ((0, pad), (0, 0)))
    pts_p = jnp.pad(pts, ((0, pad), (0, 0)))
    idx_p = jnp.pad(idx, (0, pad * K))

    wcomb = jnp.concatenate([
        Wpos[0:3] + Wpos[6:9],
        Wpos[3:6] - Wpos[6:9],
        Wpos[9:10],
        bpos[None, :],
    ], axis=0)
    zeros = jnp.zeros((NPAD, 1), jnp.float32)
    ones = jnp.ones((NPAD, 1), jnp.float32)
    ptsA = jnp.concatenate([pts_p, zeros, zeros, zeros, zeros, ones], axis=1)
    ptsB = jnp.concatenate([zeros, zeros, zeros, pts_p, zeros, zeros], axis=1)

    q, p, wcg = _prep(feats_p, ptsB, ptsA, wcomb, Wg, bg[None, :])
    xyz_t = jnp.pad(pts_p.T, ((0, 0), (0, 16)))
    m = _sc_edge(q, p, xyz_t[0], xyz_t[1], xyz_t[2], idx_p, wcg[6])
    out = _outk(m[:N], feats, Wo, bo[None, :], gamma[None, :], beta[None, :])
    return out[None]


# speedup vs baseline: 1.4406x; 1.4406x over previous
import functools

import jax
import jax.numpy as jnp
from jax import lax
from jax.experimental import pallas as pl
from jax.experimental.pallas import tpu as pltpu
from jax.experimental.pallas import tpu_sc as plsc

N = 10000
K = 32
C = 128
NW = 32
CPC = 4
CPT = 320
NPAD = NW * CPT
NCHUNK = CPT // CPC
NBUF = 2
IDXC = CPC * K
FCH = C // 16
FCH2 = C // 32
NPAD16 = NPAD + 16


def _prep_body(feats, ptsB, ptsA, wcomb, wg, bg, q_out, p_out, wcg_out):
    wcg = jnp.dot(wcomb[:], wg[:], preferred_element_type=jnp.float32)
    q = (jnp.dot(feats[:], wg[:], preferred_element_type=jnp.float32)
         + jnp.dot(ptsB[:], wcg, preferred_element_type=jnp.float32))
    q_out[:] = q.astype(jnp.bfloat16)
    p_out[:] = jnp.dot(ptsA[:], wcg, preferred_element_type=jnp.float32) + bg[:]
    wcg_out[:] = wcg.astype(jnp.bfloat16)


_prep = pl.pallas_call(
    _prep_body,
    out_shape=(
        jax.ShapeDtypeStruct((NPAD, C), jnp.bfloat16),
        jax.ShapeDtypeStruct((NPAD, C), jnp.float32),
        jax.ShapeDtypeStruct((8, C), jnp.bfloat16),
    ),
)


def _rsqrt(d2):
    i = plsc.bitcast(d2, jnp.int32)
    i = jnp.int32(0x5F3759DF) - lax.shift_right_logical(i, 1)
    r = plsc.bitcast(i, jnp.float32)
    for _ in range(3):
        r = r * (1.5 - 0.5 * d2 * r * r)
    return r


def _sc_body(q_hbm, x_hbm, y_hbm, z_hbm, idx_hbm, w9_hbm, m_hbm,
             x_v, y_v, z_v, idx_all, qb0, qb1, w9v, oall, sem0, sem1):
    wid = lax.axis_index("s") * 2 + lax.axis_index("c")
    base_center = wid * CPT

    pltpu.sync_copy(x_hbm, x_v)
    pltpu.sync_copy(y_hbm, y_v)
    pltpu.sync_copy(z_hbm, z_v)
    pltpu.sync_copy(w9_hbm, w9v)
    pltpu.sync_copy(idx_hbm.at[pl.ds(base_center * K, CPT * K)], idx_all)

    bufs = ((qb0, sem0), (qb1, sem1))
    UNFMT = dict(format=plsc.PackFormat.INTERLEAVED)

    def unpk(i32vec):
        return plsc.unpack(plsc.bitcast(i32vec, jnp.bfloat16), **UNFMT)

    wab = [unpk(w9v[pl.ds(f * 16, 16)]) for f in range(FCH2)]

    def fire(ci, qb, sem):
        pltpu.async_copy(q_hbm.at[idx_all.at[pl.ds(ci * IDXC, IDXC)]], qb, sem)

    def drain(ci, qb, sem):
        pltpu.make_async_copy(
            q_hbm.at[idx_all.at[pl.ds(ci * IDXC, IDXC)]], qb, sem).wait()

    def compute(ci, qb):
        cvec = base_center + ci * CPC
        xc = x_v[pl.ds(cvec, 16)]
        yc = y_v[pl.ds(cvec, 16)]
        zc = z_v[pl.ds(cvec, 16)]
        for cc in range(CPC):
            xi, yi, zi = xc[cc], yc[cc], zc[cc]
            dist = []
            for g in range(2):
                jv = idx_all[pl.ds(ci * IDXC + cc * K + g * 16, 16)]
                dx = xi - plsc.load_gather(x_v, [jv])
                dy = yi - plsc.load_gather(y_v, [jv])
                dz = zi - plsc.load_gather(z_v, [jv])
                d2 = dx * dx + dy * dy + dz * dz
                dist.append(d2 * _rsqrt(jnp.maximum(d2, 1e-24)))
            acca = [None] * FCH2
            accb = [None] * FCH2
            for k in range(K):
                ds_ = dist[k // 16][k % 16]
                row = cc * K + k
                for f in range(FCH2):
                    a, b = unpk(qb[row, pl.ds(f * 16, 16)])
                    va = a + ds_ * wab[f][0]
                    vb = b + ds_ * wab[f][1]
                    if k == 0:
                        acca[f], accb[f] = va, vb
                    else:
                        acca[f] = jnp.maximum(acca[f], va)
                        accb[f] = jnp.maximum(accb[f], vb)
            lrow = ci * CPC + cc
            for f in range(FCH2):
                oall[lrow, pl.ds(f * 16, 16)] = plsc.bitcast(
                    plsc.pack(acca[f], accb[f], **UNFMT), jnp.int32)

    for b in range(NBUF):
        fire(b, *bufs[b])

    def outer(cg, carry):
        for b in range(NBUF):
            ci = cg * NBUF + b
            qb, sem = bufs[b]
            drain(ci, qb, sem)
            compute(ci, qb)
            nci = ci + NBUF

            @pl.when(nci < NCHUNK)
            def _():
                fire(nci, qb, sem)
        return carry

    lax.fori_loop(0, NCHUNK // NBUF, outer, 0)
    pltpu.sync_copy(oall, m_hbm.at[pl.ds(base_center, CPT)])


_sc_edge = functools.partial(
    pl.kernel,
    mesh=plsc.VectorSubcoreMesh(core_axis_name="c", subcore_axis_name="s"),
    out_type=jax.ShapeDtypeStruct((NPAD, C // 2), jnp.int32),
    compiler_params=pltpu.CompilerParams(needs_layout_passes=False,
                                         use_tc_tiling_on_sc=False),
    scratch_types=[
        pltpu.VMEM((NPAD16,), jnp.float32),
        pltpu.VMEM((NPAD16,), jnp.float32),
        pltpu.VMEM((NPAD16,), jnp.float32),
        pltpu.VMEM((CPT * K,), jnp.int32),
        pltpu.VMEM((IDXC, C // 2), jnp.int32),
        pltpu.VMEM((IDXC, C // 2), jnp.int32),
        pltpu.VMEM((C // 2,), jnp.int32),
        pltpu.VMEM((CPT, C // 2), jnp.int32),
        pltpu.SemaphoreType.DMA,
        pltpu.SemaphoreType.DMA,
    ],
)(_sc_body)


def _out_body(m, p, feats, wo, bo, gamma, beta, o_ref):
    y = jnp.maximum(m[:].astype(jnp.float32) + p[:], 0.0) + feats[:]
    z = (jnp.dot(y, wo[:], preferred_element_type=jnp.float32)
         + bo[:])
    mu = jnp.mean(z, axis=-1, keepdims=True)
    var = jnp.mean((z - mu) ** 2, axis=-1, keepdims=True)
    o_ref[:] = jnp.maximum(
        (z - mu) / jnp.sqrt(var + 1e-5) * gamma[:] + beta[:], 0.0)


_outk = pl.pallas_call(
    _out_body,
    out_shape=jax.ShapeDtypeStruct((N, C), jnp.float32),
)


def kernel(points, features, grouped_idx, Wpos, bpos, Wg, bg, Wo, bo, gamma, beta):
    pts = points[0]
    feats = features[0]
    idx = grouped_idx[0].astype(jnp.int32).reshape(-1)

    pad = NPAD - N
    feats_p = jnp.pad(feats, ((0, pad), (0, 0)))
    pts_p = jnp.pad(pts, ((0, pad), (0, 0)))
    idx_p = jnp.pad(idx, (0, pad * K))

    wcomb = jnp.concatenate([
        Wpos[0:3] + Wpos[6:9],
        Wpos[3:6] - Wpos[6:9],
        Wpos[9:10],
        bpos[None, :],
    ], axis=0)
    zeros = jnp.zeros((NPAD, 1), jnp.float32)
    ones = jnp.ones((NPAD, 1), jnp.float32)
    ptsA = jnp.concatenate([pts_p, zeros, zeros, zeros, zeros, ones], axis=1)
    ptsB = jnp.concatenate([zeros, zeros, zeros, pts_p, zeros, zeros], axis=1)

    q, p, wcg = _prep(feats_p, ptsB, ptsA, wcomb, Wg, bg[None, :])
    q32 = jax.lax.bitcast_convert_type(q.reshape(NPAD, C // 2, 2), jnp.int32)
    w932 = jax.lax.bitcast_convert_type(wcg[6].reshape(C // 2, 2), jnp.int32)
    xyz_t = jnp.pad(pts_p.T, ((0, 0), (0, 16)))
    m32 = _sc_edge(q32, xyz_t[0], xyz_t[1], xyz_t[2], idx_p, w932)
    m = jax.lax.bitcast_convert_type(m32[:N], jnp.bfloat16).reshape(N, C)
    out = _outk(m, p[:N], feats, Wo, bo[None, :], gamma[None, :],
                beta[None, :])
    return out[None]


# speedup vs baseline: 1.5894x; 1.1033x over previous
import functools

import jax
import jax.numpy as jnp
from jax import lax
from jax.experimental import pallas as pl
from jax.experimental.pallas import tpu as pltpu
from jax.experimental.pallas import tpu_sc as plsc

N = 10000
K = 32
C = 128
NW = 32
CPC = 4
CPT = 320
NPAD = NW * CPT
NCHUNK = CPT // CPC
NBUF = 2
IDXC = CPC * K
FCH = C // 16
FCH2 = C // 32
NPAD16 = NPAD + 16


def _prep_body(feats, ptsB, ptsA, wcomb, wg, bg, q_out, p_out, wcg_out):
    wcg = jnp.dot(wcomb[:], wg[:], preferred_element_type=jnp.float32)
    q = (jnp.dot(feats[:], wg[:], preferred_element_type=jnp.float32)
         + jnp.dot(ptsB[:], wcg, preferred_element_type=jnp.float32))
    q_out[:] = q.astype(jnp.bfloat16)
    p_out[:] = jnp.dot(ptsA[:], wcg, preferred_element_type=jnp.float32) + bg[:]
    wcg_out[:] = wcg.astype(jnp.bfloat16)


_prep = pl.pallas_call(
    _prep_body,
    out_shape=(
        jax.ShapeDtypeStruct((NPAD, C), jnp.bfloat16),
        jax.ShapeDtypeStruct((NPAD, C), jnp.float32),
        jax.ShapeDtypeStruct((8, C), jnp.bfloat16),
    ),
)


def _rsqrt(d2):
    i = plsc.bitcast(d2, jnp.int32)
    i = jnp.int32(0x5F3759DF) - lax.shift_right_logical(i, 1)
    r = plsc.bitcast(i, jnp.float32)
    for _ in range(3):
        r = r * (1.5 - 0.5 * d2 * r * r)
    return r


def _sc_body(q_hbm, x_hbm, y_hbm, z_hbm, idx_hbm, w9_hbm, m_hbm,
             x_v, y_v, z_v, idx_all, qb0, qb1, w9v, oall, q_spm, sem0, sem1):
    sid = lax.axis_index("s")
    wid = sid * 2 + lax.axis_index("c")
    base_center = wid * CPT

    rpt = NPAD // 16
    pltpu.sync_copy(q_hbm.at[pl.ds(sid * rpt, rpt)],
                    q_spm.at[pl.ds(sid * rpt, rpt)])
    pltpu.sync_copy(x_hbm, x_v)
    pltpu.sync_copy(y_hbm, y_v)
    pltpu.sync_copy(z_hbm, z_v)
    pltpu.sync_copy(w9_hbm, w9v)
    pltpu.sync_copy(idx_hbm.at[pl.ds(base_center * K, CPT * K)], idx_all)
    plsc.subcore_barrier()

    bufs = ((qb0, sem0), (qb1, sem1))
    UNFMT = dict(format=plsc.PackFormat.INTERLEAVED)

    def unpk(i32vec):
        return plsc.unpack(plsc.bitcast(i32vec, jnp.bfloat16), **UNFMT)

    wab = [unpk(w9v[pl.ds(f * 16, 16)]) for f in range(FCH2)]

    def fire(ci, qb, sem):
        pltpu.async_copy(q_spm.at[idx_all.at[pl.ds(ci * IDXC, IDXC)]], qb, sem)

    def drain(ci, qb, sem):
        pltpu.make_async_copy(
            q_spm.at[idx_all.at[pl.ds(ci * IDXC, IDXC)]], qb, sem).wait()

    def compute(ci, qb):
        cvec = base_center + ci * CPC
        xc = x_v[pl.ds(cvec, 16)]
        yc = y_v[pl.ds(cvec, 16)]
        zc = z_v[pl.ds(cvec, 16)]
        for cc in range(CPC):
            xi, yi, zi = xc[cc], yc[cc], zc[cc]
            dist = []
            for g in range(2):
                jv = idx_all[pl.ds(ci * IDXC + cc * K + g * 16, 16)]
                dx = xi - plsc.load_gather(x_v, [jv])
                dy = yi - plsc.load_gather(y_v, [jv])
                dz = zi - plsc.load_gather(z_v, [jv])
                d2 = dx * dx + dy * dy + dz * dz
                dist.append(d2 * _rsqrt(jnp.maximum(d2, 1e-24)))
            acca = [None] * FCH2
            accb = [None] * FCH2
            for k in range(K):
                ds_ = dist[k // 16][k % 16]
                row = cc * K + k
                for f in range(FCH2):
                    a, b = unpk(qb[row, pl.ds(f * 16, 16)])
                    va = a + ds_ * wab[f][0]
                    vb = b + ds_ * wab[f][1]
                    if k == 0:
                        acca[f], accb[f] = va, vb
                    else:
                        acca[f] = jnp.maximum(acca[f], va)
                        accb[f] = jnp.maximum(accb[f], vb)
            lrow = ci * CPC + cc
            for f in range(FCH2):
                oall[lrow, pl.ds(f * 16, 16)] = plsc.bitcast(
                    plsc.pack(acca[f], accb[f], **UNFMT), jnp.int32)

    for b in range(NBUF):
        fire(b, *bufs[b])

    def outer(cg, carry):
        for b in range(NBUF):
            ci = cg * NBUF + b
            qb, sem = bufs[b]
            drain(ci, qb, sem)
            compute(ci, qb)
            nci = ci + NBUF

            @pl.when(nci < NCHUNK)
            def _():
                fire(nci, qb, sem)
        return carry

    lax.fori_loop(0, NCHUNK // NBUF, outer, 0)
    pltpu.sync_copy(oall, m_hbm.at[pl.ds(base_center, CPT)])


_sc_edge = functools.partial(
    pl.kernel,
    mesh=plsc.VectorSubcoreMesh(core_axis_name="c", subcore_axis_name="s"),
    out_type=jax.ShapeDtypeStruct((NPAD, C // 2), jnp.int32),
    compiler_params=pltpu.CompilerParams(needs_layout_passes=False,
                                         use_tc_tiling_on_sc=False),
    scratch_types=[
        pltpu.VMEM((NPAD16,), jnp.float32),
        pltpu.VMEM((NPAD16,), jnp.float32),
        pltpu.VMEM((NPAD16,), jnp.float32),
        pltpu.VMEM((CPT * K,), jnp.int32),
        pltpu.VMEM((IDXC, C // 2), jnp.int32),
        pltpu.VMEM((IDXC, C // 2), jnp.int32),
        pltpu.VMEM((C // 2,), jnp.int32),
        pltpu.VMEM((CPT, C // 2), jnp.int32),
        pltpu.VMEM_SHARED((NPAD, C // 2), jnp.int32),
        pltpu.SemaphoreType.DMA,
        pltpu.SemaphoreType.DMA,
    ],
)(_sc_body)


def _out_body(m, p, feats, wo, bo, gamma, beta, o_ref):
    y = jnp.maximum(m[:].astype(jnp.float32) + p[:], 0.0) + feats[:]
    z = (jnp.dot(y, wo[:], preferred_element_type=jnp.float32)
         + bo[:])
    mu = jnp.mean(z, axis=-1, keepdims=True)
    var = jnp.mean((z - mu) ** 2, axis=-1, keepdims=True)
    o_ref[:] = jnp.maximum(
        (z - mu) / jnp.sqrt(var + 1e-5) * gamma[:] + beta[:], 0.0)


_outk = pl.pallas_call(
    _out_body,
    out_shape=jax.ShapeDtypeStruct((N, C), jnp.float32),
)


def kernel(points, features, grouped_idx, Wpos, bpos, Wg, bg, Wo, bo, gamma, beta):
    pts = points[0]
    feats = features[0]
    idx = grouped_idx[0].astype(jnp.int32).reshape(-1)

    pad = NPAD - N
    feats_p = jnp.pad(feats, ((0, pad), (0, 0)))
    pts_p = jnp.pad(pts, ((0, pad), (0, 0)))
    idx_p = jnp.pad(idx, (0, pad * K))

    wcomb = jnp.concatenate([
        Wpos[0:3] + Wpos[6:9],
        Wpos[3:6] - Wpos[6:9],
        Wpos[9:10],
        bpos[None, :],
    ], axis=0)
    zeros = jnp.zeros((NPAD, 1), jnp.float32)
    ones = jnp.ones((NPAD, 1), jnp.float32)
    ptsA = jnp.concatenate([pts_p, zeros, zeros, zeros, zeros, ones], axis=1)
    ptsB = jnp.concatenate([zeros, zeros, zeros, pts_p, zeros, zeros], axis=1)

    q, p, wcg = _prep(feats_p, ptsB, ptsA, wcomb, Wg, bg[None, :])
    q32 = jax.lax.bitcast_convert_type(q.reshape(NPAD, C // 2, 2), jnp.int32)
    w932 = jax.lax.bitcast_convert_type(wcg[6].reshape(C // 2, 2), jnp.int32)
    xyz_t = jnp.pad(pts_p.T, ((0, 0), (0, 16)))
    m32 = _sc_edge(q32, xyz_t[0], xyz_t[1], xyz_t[2], idx_p, w932)
    m = jax.lax.bitcast_convert_type(m32[:N], jnp.bfloat16).reshape(N, C)
    out = _outk(m, p[:N], feats, Wo, bo[None, :], gamma[None, :],
                beta[None, :])
    return out[None]


# speedup vs baseline: 2.4277x; 1.5274x over previous
import functools

import jax
import jax.numpy as jnp
from jax import lax
from jax.experimental import pallas as pl
from jax.experimental.pallas import tpu as pltpu
from jax.experimental.pallas import tpu_sc as plsc

N = 10000
K = 32
C = 128
NW = 32
CPC = 4
CPT = 320
NPAD = NW * CPT
NCHUNK = CPT // CPC
NBUF = 2
IDXC = CPC * K
FCH = C // 16
FCH2 = C // 32
NPAD16 = NPAD + 16


def _prep_body(feats, ptsB, ptsA, wcomb, wg, bg, q_out, p_out, wcg_out):
    wcg = jnp.dot(wcomb[:], wg[:], preferred_element_type=jnp.float32)
    q = (jnp.dot(feats[:], wg[:], preferred_element_type=jnp.float32)
         + jnp.dot(ptsB[:], wcg, preferred_element_type=jnp.float32))
    q_out[:] = q.astype(jnp.bfloat16)
    p_out[:] = jnp.dot(ptsA[:], wcg, preferred_element_type=jnp.float32) + bg[:]
    wcg_out[:] = wcg.astype(jnp.bfloat16)


_prep = pl.pallas_call(
    _prep_body,
    out_shape=(
        jax.ShapeDtypeStruct((NPAD, C), jnp.bfloat16),
        jax.ShapeDtypeStruct((NPAD, C), jnp.float32),
        jax.ShapeDtypeStruct((8, C), jnp.bfloat16),
    ),
)


def _rsqrt(d2):
    i = plsc.bitcast(d2, jnp.int32)
    i = jnp.int32(0x5F3759DF) - lax.shift_right_logical(i, 1)
    r = plsc.bitcast(i, jnp.float32)
    for _ in range(3):
        r = r * (1.5 - 0.5 * d2 * r * r)
    return r


def _sc_body(q_hbm, x_hbm, y_hbm, z_hbm, idx_hbm, w9_hbm, m_hbm,
             x_v, y_v, z_v, idx_all, qb0, qb1, w9v, oall, q_spm, sem0, sem1):
    sid = lax.axis_index("s")
    wid = sid * 2 + lax.axis_index("c")
    base_center = wid * CPT

    rpt = NPAD // 16
    pltpu.sync_copy(q_hbm.at[pl.ds(sid * rpt, rpt)],
                    q_spm.at[pl.ds(sid * rpt, rpt)])
    pltpu.sync_copy(x_hbm, x_v)
    pltpu.sync_copy(y_hbm, y_v)
    pltpu.sync_copy(z_hbm, z_v)
    pltpu.sync_copy(w9_hbm, w9v)
    pltpu.sync_copy(idx_hbm.at[pl.ds(base_center * K, CPT * K)], idx_all)
    plsc.subcore_barrier()

    bufs = ((qb0, sem0), (qb1, sem1))
    wch = [plsc.bitcast(w9v[pl.ds(f * 16, 16)], jnp.bfloat16)
           for f in range(FCH2)]

    def fire(ci, qb, sem):
        pltpu.async_copy(q_spm.at[idx_all.at[pl.ds(ci * IDXC, IDXC)]], qb, sem)

    def drain(ci, qb, sem):
        pltpu.make_async_copy(
            q_spm.at[idx_all.at[pl.ds(ci * IDXC, IDXC)]], qb, sem).wait()

    def compute(ci, qb):
        cvec = base_center + ci * CPC
        xc = x_v[pl.ds(cvec, 16)]
        yc = y_v[pl.ds(cvec, 16)]
        zc = z_v[pl.ds(cvec, 16)]
        for cc in range(CPC):
            xi, yi, zi = xc[cc], yc[cc], zc[cc]
            dist = []
            for g in range(2):
                jv = idx_all[pl.ds(ci * IDXC + cc * K + g * 16, 16)]
                dx = xi - plsc.load_gather(x_v, [jv])
                dy = yi - plsc.load_gather(y_v, [jv])
                dz = zi - plsc.load_gather(z_v, [jv])
                d2 = dx * dx + dy * dy + dz * dz
                dist.append(d2 * _rsqrt(jnp.maximum(d2, 1e-24)))
            acc = [None] * FCH2
            for k in range(K):
                d16 = jnp.full((16,), dist[k // 16][k % 16], jnp.float32)
                dsb = plsc.pack(d16, d16, format=plsc.PackFormat.INTERLEAVED)
                row = cc * K + k
                for f in range(FCH2):
                    qv = plsc.bitcast(qb[row, pl.ds(f * 16, 16)],
                                      jnp.bfloat16)
                    v = qv + dsb * wch[f]
                    acc[f] = v if k == 0 else jnp.maximum(acc[f], v)
            lrow = ci * CPC + cc
            for f in range(FCH2):
                oall[lrow, pl.ds(f * 16, 16)] = plsc.bitcast(acc[f],
                                                             jnp.int32)

    for b in range(NBUF):
        fire(b, *bufs[b])

    def outer(cg, carry):
        for b in range(NBUF):
            ci = cg * NBUF + b
            qb, sem = bufs[b]
            drain(ci, qb, sem)
            compute(ci, qb)
            nci = ci + NBUF

            @pl.when(nci < NCHUNK)
            def _():
                fire(nci, qb, sem)
        return carry

    lax.fori_loop(0, NCHUNK // NBUF, outer, 0)
    pltpu.sync_copy(oall, m_hbm.at[pl.ds(base_center, CPT)])


_sc_edge = functools.partial(
    pl.kernel,
    mesh=plsc.VectorSubcoreMesh(core_axis_name="c", subcore_axis_name="s"),
    out_type=jax.ShapeDtypeStruct((NPAD, C // 2), jnp.int32),
    compiler_params=pltpu.CompilerParams(needs_layout_passes=False,
                                         use_tc_tiling_on_sc=False),
    scratch_types=[
        pltpu.VMEM((NPAD16,), jnp.float32),
        pltpu.VMEM((NPAD16,), jnp.float32),
        pltpu.VMEM((NPAD16,), jnp.float32),
        pltpu.VMEM((CPT * K,), jnp.int32),
        pltpu.VMEM((IDXC, C // 2), jnp.int32),
        pltpu.VMEM((IDXC, C // 2), jnp.int32),
        pltpu.VMEM((C // 2,), jnp.int32),
        pltpu.VMEM((CPT, C // 2), jnp.int32),
        pltpu.VMEM_SHARED((NPAD, C // 2), jnp.int32),
        pltpu.SemaphoreType.DMA,
        pltpu.SemaphoreType.DMA,
    ],
)(_sc_body)


def _out_body(m, p, feats, wo, bo, gamma, beta, o_ref):
    y = jnp.maximum(m[:].astype(jnp.float32) + p[:], 0.0) + feats[:]
    z = (jnp.dot(y, wo[:], preferred_element_type=jnp.float32)
         + bo[:])
    mu = jnp.mean(z, axis=-1, keepdims=True)
    var = jnp.mean((z - mu) ** 2, axis=-1, keepdims=True)
    o_ref[:] = jnp.maximum(
        (z - mu) / jnp.sqrt(var + 1e-5) * gamma[:] + beta[:], 0.0)


_outk = pl.pallas_call(
    _out_body,
    out_shape=jax.ShapeDtypeStruct((N, C), jnp.float32),
)


def kernel(points, features, grouped_idx, Wpos, bpos, Wg, bg, Wo, bo, gamma, beta):
    pts = points[0]
    feats = features[0]
    idx = grouped_idx[0].astype(jnp.int32).reshape(-1)

    pad = NPAD - N
    feats_p = jnp.pad(feats, ((0, pad), (0, 0)))
    pts_p = jnp.pad(pts, ((0, pad), (0, 0)))
    idx_p = jnp.pad(idx, (0, pad * K))

    wcomb = jnp.concatenate([
        Wpos[0:3] + Wpos[6:9],
        Wpos[3:6] - Wpos[6:9],
        Wpos[9:10],
        bpos[None, :],
    ], axis=0)
    zeros = jnp.zeros((NPAD, 1), jnp.float32)
    ones = jnp.ones((NPAD, 1), jnp.float32)
    ptsA = jnp.concatenate([pts_p, zeros, zeros, zeros, zeros, ones], axis=1)
    ptsB = jnp.concatenate([zeros, zeros, zeros, pts_p, zeros, zeros], axis=1)

    q, p, wcg = _prep(feats_p, ptsB, ptsA, wcomb, Wg, bg[None, :])
    q32 = jax.lax.bitcast_convert_type(q.reshape(NPAD, C // 2, 2), jnp.int32)
    w932 = jax.lax.bitcast_convert_type(wcg[6].reshape(C // 2, 2), jnp.int32)
    xyz_t = jnp.pad(pts_p.T, ((0, 0), (0, 16)))
    m32 = _sc_edge(q32, xyz_t[0], xyz_t[1], xyz_t[2], idx_p, w932)
    m = jax.lax.bitcast_convert_type(m32[:N], jnp.bfloat16).reshape(N, C)
    out = _outk(m, p[:N], feats, Wo, bo[None, :], gamma[None, :],
                beta[None, :])
    return out[None]


# speedup vs baseline: 3.1580x; 1.3008x over previous
import functools

import jax
import jax.numpy as jnp
from jax import lax
from jax.experimental import pallas as pl
from jax.experimental.pallas import tpu as pltpu
from jax.experimental.pallas import tpu_sc as plsc

N = 10000
K = 32
C = 128
NW = 32
CPC = 4
CPT = 320
NPAD = NW * CPT
NCHUNK = CPT // CPC
NBUF = 2
IDXC = CPC * K
FCH = C // 16
FCH2 = C // 32
NPAD16 = NPAD + 16


def _rne_bf16_bits(x):
    b = jax.lax.bitcast_convert_type(x, jnp.int32)
    r = b + 0x7FFF + (lax.shift_right_logical(b, 16) & 1)
    return lax.shift_right_logical(r, 16)


def _pack_pairs(x):
    lo = _rne_bf16_bits(x[:, : C // 2])
    hi = _rne_bf16_bits(x[:, C // 2:])
    return lo | lax.shift_left(hi, 16)


def _prep_body(feats, ptsB, ptsA, wcomb, wg, bg, q_out, p_out, wcg_out):
    wcg = jnp.dot(wcomb[:], wg[:], preferred_element_type=jnp.float32)
    q = (jnp.dot(feats[:], wg[:], preferred_element_type=jnp.float32)
         + jnp.dot(ptsB[:], wcg, preferred_element_type=jnp.float32))
    q_out[:] = _pack_pairs(q)
    p_out[:] = jnp.dot(ptsA[:], wcg, preferred_element_type=jnp.float32) + bg[:]
    wcg_out[:] = _pack_pairs(wcg)


_prep = pl.pallas_call(
    _prep_body,
    out_shape=(
        jax.ShapeDtypeStruct((NPAD, C // 2), jnp.int32),
        jax.ShapeDtypeStruct((NPAD, C), jnp.float32),
        jax.ShapeDtypeStruct((8, C // 2), jnp.int32),
    ),
)


def _rsqrt(d2):
    i = plsc.bitcast(d2, jnp.int32)
    i = jnp.int32(0x5F3759DF) - lax.shift_right_logical(i, 1)
    r = plsc.bitcast(i, jnp.float32)
    for _ in range(3):
        r = r * (1.5 - 0.5 * d2 * r * r)
    return r


def _sc_body(q_hbm, x_hbm, y_hbm, z_hbm, idx_hbm, w9_hbm, m_hbm,
             x_v, y_v, z_v, idx_all, qb0, qb1, w9v, oall, q_spm, sem0, sem1):
    sid = lax.axis_index("s")
    wid = sid * 2 + lax.axis_index("c")
    base_center = wid * CPT

    rpt = NPAD // 16
    pltpu.sync_copy(q_hbm.at[pl.ds(sid * rpt, rpt)],
                    q_spm.at[pl.ds(sid * rpt, rpt)])
    pltpu.sync_copy(x_hbm, x_v)
    pltpu.sync_copy(y_hbm, y_v)
    pltpu.sync_copy(z_hbm, z_v)
    pltpu.sync_copy(w9_hbm, w9v)
    pltpu.sync_copy(idx_hbm.at[pl.ds(base_center * K, CPT * K)], idx_all)
    plsc.subcore_barrier()

    bufs = ((qb0, sem0), (qb1, sem1))
    wch = [plsc.bitcast(w9v[pl.ds(f * 16, 16)], jnp.bfloat16)
           for f in range(FCH2)]

    def fire(ci, qb, sem):
        pltpu.async_copy(q_spm.at[idx_all.at[pl.ds(ci * IDXC, IDXC)]], qb, sem)

    def drain(ci, qb, sem):
        pltpu.make_async_copy(
            q_spm.at[idx_all.at[pl.ds(ci * IDXC, IDXC)]], qb, sem).wait()

    def compute(ci, qb):
        cvec = base_center + ci * CPC
        xc = x_v[pl.ds(cvec, 16)]
        yc = y_v[pl.ds(cvec, 16)]
        zc = z_v[pl.ds(cvec, 16)]
        for cc in range(CPC):
            xi, yi, zi = xc[cc], yc[cc], zc[cc]
            dist = []
            for g in range(2):
                jv = idx_all[pl.ds(ci * IDXC + cc * K + g * 16, 16)]
                dx = xi - plsc.load_gather(x_v, [jv])
                dy = yi - plsc.load_gather(y_v, [jv])
                dz = zi - plsc.load_gather(z_v, [jv])
                d2 = dx * dx + dy * dy + dz * dz
                dist.append(d2 * _rsqrt(jnp.maximum(d2, 1e-24)))
            acc = [None] * FCH2
            for k in range(K):
                d16 = jnp.full((16,), dist[k // 16][k % 16], jnp.float32)
                dsb = plsc.pack(d16, d16, format=plsc.PackFormat.INTERLEAVED)
                row = cc * K + k
                for f in range(FCH2):
                    qv = plsc.bitcast(qb[row, pl.ds(f * 16, 16)],
                                      jnp.bfloat16)
                    v = qv + dsb * wch[f]
                    acc[f] = v if k == 0 else jnp.maximum(acc[f], v)
            lrow = ci * CPC + cc
            for f in range(FCH2):
                oall[lrow, pl.ds(f * 16, 16)] = plsc.bitcast(acc[f],
                                                             jnp.int32)

    for b in range(NBUF):
        fire(b, *bufs[b])

    def outer(cg, carry):
        for b in range(NBUF):
            ci = cg * NBUF + b
            qb, sem = bufs[b]
            drain(ci, qb, sem)
            compute(ci, qb)
            nci = ci + NBUF

            @pl.when(nci < NCHUNK)
            def _():
                fire(nci, qb, sem)
        return carry

    lax.fori_loop(0, NCHUNK // NBUF, outer, 0)
    pltpu.sync_copy(oall, m_hbm.at[pl.ds(base_center, CPT)])


_sc_edge = functools.partial(
    pl.kernel,
    mesh=plsc.VectorSubcoreMesh(core_axis_name="c", subcore_axis_name="s"),
    out_type=jax.ShapeDtypeStruct((NPAD, C // 2), jnp.int32),
    compiler_params=pltpu.CompilerParams(needs_layout_passes=False,
                                         use_tc_tiling_on_sc=False),
    scratch_types=[
        pltpu.VMEM((NPAD16,), jnp.float32),
        pltpu.VMEM((NPAD16,), jnp.float32),
        pltpu.VMEM((NPAD16,), jnp.float32),
        pltpu.VMEM((CPT * K,), jnp.int32),
        pltpu.VMEM((IDXC, C // 2), jnp.int32),
        pltpu.VMEM((IDXC, C // 2), jnp.int32),
        pltpu.VMEM((C // 2,), jnp.int32),
        pltpu.VMEM((CPT, C // 2), jnp.int32),
        pltpu.VMEM_SHARED((NPAD, C // 2), jnp.int32),
        pltpu.SemaphoreType.DMA,
        pltpu.SemaphoreType.DMA,
    ],
)(_sc_body)


def _out_body(m32, p, feats, wo, bo, gamma, beta, o_ref):
    w = m32[:]
    lo = jax.lax.bitcast_convert_type(lax.shift_left(w, 16), jnp.float32)
    hi = jax.lax.bitcast_convert_type(
        w & jnp.int32(-65536), jnp.float32)
    m = jnp.concatenate([lo, hi], axis=1)
    y = jnp.maximum(m + p[:], 0.0) + feats[:]
    z = (jnp.dot(y, wo[:], preferred_element_type=jnp.float32)
         + bo[:])
    mu = jnp.mean(z, axis=-1, keepdims=True)
    var = jnp.mean((z - mu) ** 2, axis=-1, keepdims=True)
    o_ref[:] = jnp.maximum(
        (z - mu) / jnp.sqrt(var + 1e-5) * gamma[:] + beta[:], 0.0)


_outk = pl.pallas_call(
    _out_body,
    out_shape=jax.ShapeDtypeStruct((N, C), jnp.float32),
)


def kernel(points, features, grouped_idx, Wpos, bpos, Wg, bg, Wo, bo, gamma, beta):
    pts = points[0]
    feats = features[0]
    idx = grouped_idx[0].astype(jnp.int32).reshape(-1)

    pad = NPAD - N
    feats_p = jnp.pad(feats, ((0, pad), (0, 0)))
    pts_p = jnp.pad(pts, ((0, pad), (0, 0)))
    idx_p = jnp.pad(idx, (0, pad * K))

    wcomb = jnp.concatenate([
        Wpos[0:3] + Wpos[6:9],
        Wpos[3:6] - Wpos[6:9],
        Wpos[9:10],
        bpos[None, :],
    ], axis=0)
    zeros = jnp.zeros((NPAD, 1), jnp.float32)
    ones = jnp.ones((NPAD, 1), jnp.float32)
    ptsA = jnp.concatenate([pts_p, zeros, zeros, zeros, zeros, ones], axis=1)
    ptsB = jnp.concatenate([zeros, zeros, zeros, pts_p, zeros, zeros], axis=1)

    q32, p, wcg32 = _prep(feats_p, ptsB, ptsA, wcomb, Wg, bg[None, :])
    xyz_t = jnp.pad(pts_p.T, ((0, 0), (0, 16)))
    m32 = _sc_edge(q32, xyz_t[0], xyz_t[1], xyz_t[2], idx_p, wcg32[6])
    out = _outk(m32[:N], p[:N], feats, Wo, bo[None, :], gamma[None, :],
                beta[None, :])
    return out[None]


# speedup vs baseline: 3.2749x; 1.0370x over previous
import functools

import jax
import jax.numpy as jnp
from jax import lax
from jax.experimental import pallas as pl
from jax.experimental.pallas import tpu as pltpu
from jax.experimental.pallas import tpu_sc as plsc

N = 10000
K = 32
C = 128
NW = 32
CPC = 4
CPT = 320
NPAD = NW * CPT
NCHUNK = CPT // CPC
NBUF = 2
IDXC = CPC * K
FCH = C // 16
FCH2 = C // 32
NPAD16 = NPAD + 16


def _rne_bf16_bits(x):
    b = jax.lax.bitcast_convert_type(x, jnp.int32)
    r = b + 0x7FFF + (lax.shift_right_logical(b, 16) & 1)
    return lax.shift_right_logical(r, 16)


def _pack_pairs(x):
    lo = _rne_bf16_bits(x[:, : C // 2])
    hi = _rne_bf16_bits(x[:, C // 2:])
    return lo | lax.shift_left(hi, 16)


def _prep_body(feats, ptsB, ptsA, wcomb, wg, bg, q_out, p_out, wcg_out):
    wcg = jnp.dot(wcomb[:], wg[:], preferred_element_type=jnp.float32)
    q = (jnp.dot(feats[:], wg[:], preferred_element_type=jnp.float32)
         + jnp.dot(ptsB[:], wcg, preferred_element_type=jnp.float32))
    q_out[0:N] = _pack_pairs(q)
    p_out[0:N] = (jnp.dot(ptsA[:], wcg, preferred_element_type=jnp.float32)
                  + bg[:])
    wcg_out[:] = _pack_pairs(wcg)


_prep = pl.pallas_call(
    _prep_body,
    out_shape=(
        jax.ShapeDtypeStruct((NPAD, C // 2), jnp.int32),
        jax.ShapeDtypeStruct((NPAD, C), jnp.float32),
        jax.ShapeDtypeStruct((8, C // 2), jnp.int32),
    ),
)


def _rsqrt(d2):
    i = plsc.bitcast(d2, jnp.int32)
    i = jnp.int32(0x5F3759DF) - lax.shift_right_logical(i, 1)
    r = plsc.bitcast(i, jnp.float32)
    for _ in range(3):
        r = r * (1.5 - 0.5 * d2 * r * r)
    return r


def _sc_body(q_hbm, x_hbm, y_hbm, z_hbm, idx_hbm, w9_hbm, m_hbm,
             x_v, y_v, z_v, idx_all, qb0, qb1, w9v, oall, q_spm, sem0, sem1):
    sid = lax.axis_index("s")
    wid = sid * 2 + lax.axis_index("c")
    base_center = wid * CPT

    rpt = NPAD // 16
    pltpu.sync_copy(q_hbm.at[pl.ds(sid * rpt, rpt)],
                    q_spm.at[pl.ds(sid * rpt, rpt)])
    pltpu.sync_copy(x_hbm, x_v)
    pltpu.sync_copy(y_hbm, y_v)
    pltpu.sync_copy(z_hbm, z_v)
    pltpu.sync_copy(w9_hbm, w9v)
    pltpu.sync_copy(idx_hbm.at[pl.ds(base_center * K, CPT * K)], idx_all)
    plsc.subcore_barrier()

    bufs = ((qb0, sem0), (qb1, sem1))
    wch = [plsc.bitcast(w9v[pl.ds(f * 16, 16)], jnp.bfloat16)
           for f in range(FCH2)]

    def fire(ci, qb, sem):
        pltpu.async_copy(q_spm.at[idx_all.at[pl.ds(ci * IDXC, IDXC)]], qb, sem)

    def drain(ci, qb, sem):
        pltpu.make_async_copy(
            q_spm.at[idx_all.at[pl.ds(ci * IDXC, IDXC)]], qb, sem).wait()

    def compute(ci, qb):
        cvec = base_center + ci * CPC
        xc = x_v[pl.ds(cvec, 16)]
        yc = y_v[pl.ds(cvec, 16)]
        zc = z_v[pl.ds(cvec, 16)]
        for cc in range(CPC):
            xi, yi, zi = xc[cc], yc[cc], zc[cc]
            dist = []
            for g in range(2):
                jv = idx_all[pl.ds(ci * IDXC + cc * K + g * 16, 16)]
                dx = xi - plsc.load_gather(x_v, [jv])
                dy = yi - plsc.load_gather(y_v, [jv])
                dz = zi - plsc.load_gather(z_v, [jv])
                d2 = dx * dx + dy * dy + dz * dz
                dist.append(d2 * _rsqrt(jnp.maximum(d2, 1e-24)))
            acc = [None] * FCH2
            for k in range(K):
                d16 = jnp.full((16,), dist[k // 16][k % 16], jnp.float32)
                dsb = plsc.pack(d16, d16, format=plsc.PackFormat.INTERLEAVED)
                row = cc * K + k
                for f in range(FCH2):
                    qv = plsc.bitcast(qb[row, pl.ds(f * 16, 16)],
                                      jnp.bfloat16)
                    v = qv + dsb * wch[f]
                    acc[f] = v if k == 0 else jnp.maximum(acc[f], v)
            lrow = ci * CPC + cc
            for f in range(FCH2):
                oall[lrow, pl.ds(f * 16, 16)] = plsc.bitcast(acc[f],
                                                             jnp.int32)

    for b in range(NBUF):
        fire(b, *bufs[b])

    def outer(cg, carry):
        for b in range(NBUF):
            ci = cg * NBUF + b
            qb, sem = bufs[b]
            drain(ci, qb, sem)
            compute(ci, qb)
            nci = ci + NBUF

            @pl.when(nci < NCHUNK)
            def _():
                fire(nci, qb, sem)
        return carry

    lax.fori_loop(0, NCHUNK // NBUF, outer, 0)
    pltpu.sync_copy(oall, m_hbm.at[pl.ds(base_center, CPT)])


_sc_edge = functools.partial(
    pl.kernel,
    mesh=plsc.VectorSubcoreMesh(core_axis_name="c", subcore_axis_name="s"),
    out_type=jax.ShapeDtypeStruct((NPAD, C // 2), jnp.int32),
    compiler_params=pltpu.CompilerParams(needs_layout_passes=False,
                                         use_tc_tiling_on_sc=False),
    scratch_types=[
        pltpu.VMEM((NPAD16,), jnp.float32),
        pltpu.VMEM((NPAD16,), jnp.float32),
        pltpu.VMEM((NPAD16,), jnp.float32),
        pltpu.VMEM((CPT * K,), jnp.int32),
        pltpu.VMEM((IDXC, C // 2), jnp.int32),
        pltpu.VMEM((IDXC, C // 2), jnp.int32),
        pltpu.VMEM((C // 2,), jnp.int32),
        pltpu.VMEM((CPT, C // 2), jnp.int32),
        pltpu.VMEM_SHARED((NPAD, C // 2), jnp.int32),
        pltpu.SemaphoreType.DMA,
        pltpu.SemaphoreType.DMA,
    ],
)(_sc_body)


def _out_body(m32, p, feats, wo, bo, gamma, beta, o_ref):
    w = m32[0:N]
    lo = jax.lax.bitcast_convert_type(lax.shift_left(w, 16), jnp.float32)
    hi = jax.lax.bitcast_convert_type(
        w & jnp.int32(-65536), jnp.float32)
    m = jnp.concatenate([lo, hi], axis=1)
    y = jnp.maximum(m + p[0:N], 0.0) + feats[:]
    z = (jnp.dot(y, wo[:], preferred_element_type=jnp.float32)
         + bo[:])
    mu = jnp.mean(z, axis=-1, keepdims=True)
    var = jnp.mean((z - mu) ** 2, axis=-1, keepdims=True)
    o_ref[:] = jnp.maximum(
        (z - mu) / jnp.sqrt(var + 1e-5) * gamma[:] + beta[:], 0.0)


_outk = pl.pallas_call(
    _out_body,
    out_shape=jax.ShapeDtypeStruct((N, C), jnp.float32),
)


def kernel(points, features, grouped_idx, Wpos, bpos, Wg, bg, Wo, bo, gamma, beta):
    pts = points[0]
    feats = features[0]
    idx = grouped_idx[0].astype(jnp.int32).reshape(-1)

    pad = NPAD - N
    idx_p = jnp.pad(idx, (0, pad * K))

    wcomb = jnp.concatenate([
        Wpos[0:3] + Wpos[6:9],
        Wpos[3:6] - Wpos[6:9],
        Wpos[9:10],
        bpos[None, :],
    ], axis=0)
    zeros = jnp.zeros((N, 1), jnp.float32)
    ones = jnp.ones((N, 1), jnp.float32)
    ptsA = jnp.concatenate([pts, zeros, zeros, zeros, zeros, ones], axis=1)
    ptsB = jnp.concatenate([zeros, zeros, zeros, pts, zeros, zeros], axis=1)

    q32, p, wcg32 = _prep(feats, ptsB, ptsA, wcomb, Wg, bg[None, :])
    xyz_t = jnp.pad(pts.T, ((0, 0), (0, NPAD16 - N)))
    m32 = _sc_edge(q32, xyz_t[0], xyz_t[1], xyz_t[2], idx_p, wcg32[6])
    out = _outk(m32, p, feats, Wo, bo[None, :], gamma[None, :],
                beta[None, :])
    return out[None]
